# Initial kernel scaffold; baseline (speedup 1.0000x reference)
import jax
import jax.numpy as jnp
from jax.experimental import pallas as pl


def kernel(x, w1, b1, w2, b2):
    raise NotImplementedError("write your pallas kernel here")


# speedup vs baseline: 1.1944x; 1.1944x over previous
import jax
import jax.numpy as jnp
from jax.experimental import pallas as pl
from jax.experimental.pallas import tpu as pltpu

_VMEM_BUDGET = 28 * 1024 * 1024


def _fused_se_kernel(x_ref, w1_ref, b1_ref, w2_ref, b2_ref, o_ref, *, inv_hw):
    sq = jnp.sum(x_ref[...].astype(jnp.float32), axis=2) * inv_hw
    h = jnp.dot(sq, w1_ref[...], preferred_element_type=jnp.float32)
    h = jnp.maximum(h + b1_ref[...], 0.0)
    e = jnp.dot(h, w2_ref[...], preferred_element_type=jnp.float32)
    e = jax.nn.sigmoid(e + b2_ref[...])
    o_ref[...] = jnp.maximum(x_ref[...] * e[:, :, None].astype(o_ref.dtype), 0.0)


def _pick_block_batch(B, C, HW, itemsize):
    hw_padded = -(-HW // 128) * 128
    per_row = C * hw_padded * itemsize
    bt_cap = max(1, _VMEM_BUDGET // (4 * per_row))
    bt = min(B, bt_cap, 8)
    while B % bt != 0:
        bt -= 1
    return bt


def kernel(x, w1, b1, w2, b2):
    B, C, H, W = x.shape
    HW = H * W
    Ch = w1.shape[1]
    itemsize = jnp.dtype(x.dtype).itemsize

    x_flat = x.reshape(B, C, HW)
    bt = _pick_block_batch(B, C, HW, itemsize)
    grid = (B // bt,)

    import functools
    body = functools.partial(_fused_se_kernel, inv_hw=1.0 / HW)

    out_flat = pl.pallas_call(
        body,
        out_shape=jax.ShapeDtypeStruct((B, C, HW), x.dtype),
        grid=grid,
        in_specs=[
            pl.BlockSpec((bt, C, HW), lambda i: (i, 0, 0)),
            pl.BlockSpec((C, Ch), lambda i: (0, 0)),
            pl.BlockSpec((1, Ch), lambda i: (0, 0)),
            pl.BlockSpec((Ch, C), lambda i: (0, 0)),
            pl.BlockSpec((1, C), lambda i: (0, 0)),
        ],
        out_specs=pl.BlockSpec((bt, C, HW), lambda i: (i, 0, 0)),
        compiler_params=pltpu.CompilerParams(
            dimension_semantics=("parallel",),
            vmem_limit_bytes=_VMEM_BUDGET,
        ),
        cost_estimate=pl.CostEstimate(
            flops=3 * B * C * HW + 4 * B * C * Ch,
            transcendentals=B * C,
            bytes_accessed=2 * B * C * HW * itemsize,
        ),
    )(x_flat, w1, b1.reshape(1, Ch), w2, b2.reshape(1, C))

    return out_flat.reshape(B, C, H, W)


# speedup vs baseline: 1.2044x; 1.0084x over previous
import jax
import jax.numpy as jnp
from jax.experimental import pallas as pl
from jax.experimental.pallas import tpu as pltpu

_VMEM_BUDGET = 56 * 1024 * 1024


def _fused_se_kernel(x_ref, w1_ref, b1_ref, w2_ref, b2_ref, o_ref, *, inv_hw):
    sq = jnp.sum(x_ref[...].astype(jnp.float32), axis=2) * inv_hw
    h = jnp.dot(sq, w1_ref[...], preferred_element_type=jnp.float32)
    h = jnp.maximum(h + b1_ref[...], 0.0)
    e = jnp.dot(h, w2_ref[...], preferred_element_type=jnp.float32)
    e = jax.nn.sigmoid(e + b2_ref[...])
    o_ref[...] = jnp.maximum(x_ref[...] * e[:, :, None].astype(o_ref.dtype), 0.0)


def _pick_block_batch(B, C, HW, itemsize):
    hw_padded = -(-HW // 128) * 128
    per_row = C * hw_padded * itemsize
    bt_cap = max(1, _VMEM_BUDGET // (4 * per_row))
    bt = min(B, bt_cap, 16)
    while B % bt != 0:
        bt -= 1
    return bt


def kernel(x, w1, b1, w2, b2):
    B, C, H, W = x.shape
    HW = H * W
    Ch = w1.shape[1]
    itemsize = jnp.dtype(x.dtype).itemsize

    x_flat = x.reshape(B, C, HW)
    bt = _pick_block_batch(B, C, HW, itemsize)
    grid = (B // bt,)

    import functools
    body = functools.partial(_fused_se_kernel, inv_hw=1.0 / HW)

    out_flat = pl.pallas_call(
        body,
        out_shape=jax.ShapeDtypeStruct((B, C, HW), x.dtype),
        grid=grid,
        in_specs=[
            pl.BlockSpec((bt, C, HW), lambda i: (i, 0, 0)),
            pl.BlockSpec((C, Ch), lambda i: (0, 0)),
            pl.BlockSpec((1, Ch), lambda i: (0, 0)),
            pl.BlockSpec((Ch, C), lambda i: (0, 0)),
            pl.BlockSpec((1, C), lambda i: (0, 0)),
        ],
        out_specs=pl.BlockSpec((bt, C, HW), lambda i: (i, 0, 0)),
        compiler_params=pltpu.CompilerParams(
            dimension_semantics=("parallel",),
            vmem_limit_bytes=_VMEM_BUDGET,
        ),
        cost_estimate=pl.CostEstimate(
            flops=3 * B * C * HW + 4 * B * C * Ch,
            transcendentals=B * C,
            bytes_accessed=2 * B * C * HW * itemsize,
        ),
    )(x_flat, w1, b1.reshape(1, Ch), w2, b2.reshape(1, C))

    return out_flat.reshape(B, C, H, W)
